# Initial kernel scaffold; baseline (speedup 1.0000x reference)
import jax
import jax.numpy as jnp
from jax.experimental import pallas as pl


def kernel(x, emb, W1, b1, W2, b2):
    raise NotImplementedError("write your pallas kernel here")


# speedup vs baseline: 15.3832x; 15.3832x over previous
import functools

import jax
import jax.numpy as jnp
from jax import lax
from jax.experimental import pallas as pl
from jax.experimental.pallas import tpu as pltpu
from jax.experimental.pallas import tpu_sc as plsc

VOCAB = 1000000
EMBED = 32
HIDDEN = 128
NUM_CLASSES = 10
BATCH = 16384
SEQ = 200

NC = 2
NS = 16
NW = NC * NS
ROWS_PER_W = BATCH // NW
CB = 4
IDX_PER_STEP = CB * SEQ
G = 800
NG = IDX_PER_STEP // G
STEPS = ROWS_PER_W // CB


PAIRS = STEPS // 2


def _pool_body(x_hbm, emb_hbm, out_hbm, idx_v, rows_v, out_v, sem0, sem1):
    wid = lax.axis_index("s") * NC + lax.axis_index("c")
    base_row = wid * ROWS_PER_W
    zero = jnp.zeros((16,), jnp.float32)
    inv = jnp.float32(1.0 / SEQ)
    sems = (sem0, sem1)

    def issue(slot, i_step):
        idx_off = (base_row + i_step * CB) * SEQ
        pltpu.sync_copy(x_hbm.at[pl.ds(idx_off, IDX_PER_STEP)], idx_v.at[slot])
        for g in range(NG):
            pltpu.async_copy(
                emb_hbm.at[idx_v.at[slot, pl.ds(g * G, G)]],
                rows_v.at[slot, pl.ds(g * G, G)],
                sems[slot],
            )

    def drain(slot):
        for g in range(NG):
            pltpu.make_async_copy(
                emb_hbm.at[idx_v.at[slot, pl.ds(g * G, G)]],
                rows_v.at[slot, pl.ds(g * G, G)],
                sems[slot],
            ).wait()

    def reduce(slot, i_step):
        def red(j, accs):
            out = []
            for r in range(CB):
                a0, a1 = accs[2 * r], accs[2 * r + 1]
                j2 = 2 * j
                a0 = a0 + rows_v[slot, r * SEQ + j2, pl.ds(0, 16)]
                a1 = a1 + rows_v[slot, r * SEQ + j2, pl.ds(16, 16)]
                a0 = a0 + rows_v[slot, r * SEQ + j2 + 1, pl.ds(0, 16)]
                a1 = a1 + rows_v[slot, r * SEQ + j2 + 1, pl.ds(16, 16)]
                out.append(a0)
                out.append(a1)
            return tuple(out)

        accs = lax.fori_loop(0, SEQ // 2, red, (zero,) * (2 * CB))
        for r in range(CB):
            obase = (i_step * CB + r) * EMBED
            out_v[pl.ds(obase, 16)] = accs[2 * r] * inv
            out_v[pl.ds(obase + 16, 16)] = accs[2 * r + 1] * inv

    issue(0, 0)

    def pair(k, carry):
        issue(1, 2 * k + 1)
        drain(0)
        reduce(0, 2 * k)

        @pl.when(k < PAIRS - 1)
        def _():
            issue(0, 2 * k + 2)

        drain(1)
        reduce(1, 2 * k + 1)
        return carry

    lax.fori_loop(0, PAIRS, pair, 0)
    pltpu.sync_copy(
        out_v, out_hbm.at[pl.ds(base_row * EMBED, ROWS_PER_W * EMBED)]
    )


_pool = functools.partial(
    pl.kernel,
    out_type=jax.ShapeDtypeStruct((BATCH * EMBED,), jnp.float32),
    mesh=plsc.VectorSubcoreMesh(core_axis_name="c", subcore_axis_name="s"),
    scratch_types=[
        pltpu.VMEM((2, IDX_PER_STEP), jnp.int32),
        pltpu.VMEM((2, IDX_PER_STEP, EMBED), jnp.float32),
        pltpu.VMEM((ROWS_PER_W * EMBED,), jnp.float32),
        pltpu.SemaphoreType.DMA,
        pltpu.SemaphoreType.DMA,
    ],
    compiler_params=pltpu.CompilerParams(use_tc_tiling_on_sc=False),
)(_pool_body)


BM = 1024


def _mlp_body(h_ref, w1_ref, b1_ref, w2_ref, b2_ref, o_ref):
    h = h_ref[...]
    z = jnp.dot(h, w1_ref[...], preferred_element_type=jnp.float32)
    z = jnp.maximum(z + b1_ref[...], 0.0)
    o_ref[...] = (
        jnp.dot(z, w2_ref[...], preferred_element_type=jnp.float32)
        + b2_ref[...]
    )


_mlp = pl.pallas_call(
    _mlp_body,
    grid=(BATCH // BM,),
    in_specs=[
        pl.BlockSpec((BM, EMBED), lambda i: (i, 0)),
        pl.BlockSpec((EMBED, HIDDEN), lambda i: (0, 0)),
        pl.BlockSpec((1, HIDDEN), lambda i: (0, 0)),
        pl.BlockSpec((HIDDEN, 128), lambda i: (0, 0)),
        pl.BlockSpec((1, 128), lambda i: (0, 0)),
    ],
    out_specs=pl.BlockSpec((BM, 128), lambda i: (i, 0)),
    out_shape=jax.ShapeDtypeStruct((BATCH, 128), jnp.float32),
)


def kernel(x, emb, W1, b1, W2, b2):
    pooled = _pool(x.reshape(-1), emb).reshape(BATCH, EMBED)
    w2p = jnp.zeros((HIDDEN, 128), jnp.float32).at[:, :NUM_CLASSES].set(W2)
    b2p = jnp.zeros((128,), jnp.float32).at[:NUM_CLASSES].set(b2)
    out = _mlp(pooled, W1, b1.reshape(1, HIDDEN), w2p, b2p.reshape(1, 128))
    return out[:, :NUM_CLASSES]
